# Initial kernel scaffold; baseline (speedup 1.0000x reference)
import jax
import jax.numpy as jnp
from jax.experimental import pallas as pl


def kernel(inputs, epsilon):
    raise NotImplementedError("write your pallas kernel here")


# speedup vs baseline: 1.3687x; 1.3687x over previous
import functools

import jax
import jax.numpy as jnp
from jax import lax
from jax.experimental import pallas as pl
from jax.experimental.pallas import tpu as pltpu
from jax.experimental.pallas import tpu_sc as plsc

L = 64
M = 64
BATCH = 4096
NUP = 33
ROWS = 2 * L * NUP * NUP
S_STRIDE = L * NUP * NUP
I_STRIDE = NUP * NUP + 1
NU_STRIDE = NUP - 1

_NC, _NS, _NL = 2, 16, 16
NW = _NC * _NS
SPW = BATCH // NW
GRP = SPW // 16


def _idx_body(in_ref, idx_ref):
    s = in_ref[...].astype(jnp.float32)
    row = lax.broadcasted_iota(jnp.int32, (L, L), 0)
    col = lax.broadcasted_iota(jnp.int32, (L, L), 1)
    tri = (row < col).astype(jnp.float32)
    nu = jax.lax.dot(s, tri, precision=jax.lax.Precision.HIGHEST)
    site = lax.broadcasted_iota(jnp.int32, (BATCH, L), 1).astype(jnp.float32)
    idx = s * S_STRIDE + site * I_STRIDE + nu * NU_STRIDE
    idx_ref[...] = idx.astype(jnp.int32)


def _sc_body(table_hbm, idx_hbm, out_hbm, idx_v, rows_v, tmp_v, out_v, sem):
    wid = lax.axis_index("s") * _NC + lax.axis_index("c")
    base = wid * SPW
    pltpu.sync_copy(idx_hbm.at[pl.ds(base, SPW)], idx_v)
    iota = lax.iota(jnp.int32, 16)

    def group(g, _):
        def sample(u, _):
            t = g * 16 + u
            pltpu.async_copy(table_hbm.at[idx_v.at[t]], rows_v, sem).wait()

            def prod(j, accs):
                return tuple(
                    accs[k] * rows_v[j, pl.ds(16 * k, 16)]
                    for k in range(M // 16))

            ones = jnp.ones((16,), jnp.float32)
            accs = lax.fori_loop(0, L, prod, (ones, ones, ones, ones))
            tmp_v[pl.ds(u * 16, 16)] = accs[0] + accs[1] + accs[2] + accs[3]
            return 0

        lax.fori_loop(0, 16, sample, 0)
        acc = jnp.zeros((16,), jnp.float32)
        for j in range(16):
            acc = acc + plsc.load_gather(tmp_v, [iota * 16 + j])
        out_v[pl.ds(g * 16, 16)] = acc
        return 0

    lax.fori_loop(0, GRP, group, 0)
    pltpu.sync_copy(out_v, out_hbm.at[pl.ds(base, SPW)])


@jax.jit
def _seg_gps(table, inputs_i32):
    idx = pl.pallas_call(
        _idx_body,
        out_shape=jax.ShapeDtypeStruct((BATCH, L), jnp.int32),
    )(inputs_i32)
    mesh = plsc.VectorSubcoreMesh(core_axis_name="c", subcore_axis_name="s")
    return pl.kernel(
        _sc_body,
        mesh=mesh,
        compiler_params=pltpu.CompilerParams(
            needs_layout_passes=False, use_tc_tiling_on_sc=False),
        out_type=jax.ShapeDtypeStruct((BATCH,), jnp.float32),
        scratch_types=[
            pltpu.VMEM((SPW, L), jnp.int32),
            pltpu.VMEM((L, M), jnp.float32),
            pltpu.VMEM((256,), jnp.float32),
            pltpu.VMEM((SPW,), jnp.float32),
            pltpu.SemaphoreType.DMA,
        ],
    )(table, idx)


def kernel(inputs, epsilon):
    table = jnp.transpose(epsilon, (0, 2, 3, 4, 1)).reshape(ROWS, M)
    return _seg_gps(table, inputs.astype(jnp.int32))


# speedup vs baseline: 1.5984x; 1.1678x over previous
import functools

import jax
import jax.numpy as jnp
from jax import lax
from jax.experimental import pallas as pl
from jax.experimental.pallas import tpu as pltpu
from jax.experimental.pallas import tpu_sc as plsc

L = 64
M = 64
BATCH = 4096
NUP = 33
ROWS = 2 * L * NUP * NUP
S_STRIDE = L * NUP * NUP
I_STRIDE = NUP * NUP + 1
NU_STRIDE = NUP - 1

_NC, _NS, _NL = 2, 16, 16
NW = _NC * _NS
SPW = BATCH // NW
GRP = SPW // 16


def _idx_body(in_ref, idx_ref):
    s = in_ref[...].astype(jnp.float32)
    row = lax.broadcasted_iota(jnp.int32, (L, L), 0)
    col = lax.broadcasted_iota(jnp.int32, (L, L), 1)
    tri = (row < col).astype(jnp.float32)
    nu = jax.lax.dot(s, tri, precision=jax.lax.Precision.HIGHEST)
    site = lax.broadcasted_iota(jnp.int32, (BATCH, L), 1).astype(jnp.float32)
    idx = s * S_STRIDE + site * I_STRIDE + nu * NU_STRIDE
    idx_ref[...] = idx.astype(jnp.int32)


def _sc_body(table_hbm, idx_hbm, out_hbm, idx_v, rows0, rows1, tmp_v, out_v,
             sem0, sem1):
    wid = lax.axis_index("s") * _NC + lax.axis_index("c")
    base = wid * SPW
    pltpu.sync_copy(idx_hbm.at[pl.ds(base, SPW)], idx_v)
    iota = lax.iota(jnp.int32, 16)

    def product(rows_v):
        def prod(j, accs):
            return tuple(
                accs[k] * rows_v[2 * j, pl.ds(16 * k, 16)] for k in range(4)
            ) + tuple(
                accs[4 + k] * rows_v[2 * j + 1, pl.ds(16 * k, 16)]
                for k in range(4))

        ones = jnp.ones((16,), jnp.float32)
        accs = lax.fori_loop(0, L // 2, prod, (ones,) * 8)
        return (accs[0] * accs[4] + accs[1] * accs[5]
                + accs[2] * accs[6] + accs[3] * accs[7])

    pltpu.async_copy(table_hbm.at[idx_v.at[0]], rows0, sem0)

    def group(g, _):
        def pair(hh, _):
            h = g * 8 + hh
            t0 = 2 * h
            pltpu.async_copy(table_hbm.at[idx_v.at[t0 + 1]], rows1, sem1)
            pltpu.make_async_copy(
                table_hbm.at[idx_v.at[t0]], rows0, sem0).wait()
            tot0 = product(rows0)

            @pl.when(h < SPW // 2 - 1)
            def _():
                pltpu.async_copy(
                    table_hbm.at[idx_v.at[t0 + 2]], rows0, sem0)

            pltpu.make_async_copy(
                table_hbm.at[idx_v.at[t0 + 1]], rows1, sem1).wait()
            tot1 = product(rows1)
            tmp_v[pl.ds((2 * hh) * 16, 16)] = tot0
            tmp_v[pl.ds((2 * hh + 1) * 16, 16)] = tot1
            return 0

        lax.fori_loop(0, 8, pair, 0)
        acc = jnp.zeros((16,), jnp.float32)
        for j in range(16):
            acc = acc + plsc.load_gather(tmp_v, [iota * 16 + j])
        out_v[pl.ds(g * 16, 16)] = acc
        return 0

    lax.fori_loop(0, GRP, group, 0)
    pltpu.sync_copy(out_v, out_hbm.at[pl.ds(base, SPW)])


@jax.jit
def _seg_gps(table, inputs_i32):
    idx = pl.pallas_call(
        _idx_body,
        out_shape=jax.ShapeDtypeStruct((BATCH, L), jnp.int32),
    )(inputs_i32)
    mesh = plsc.VectorSubcoreMesh(core_axis_name="c", subcore_axis_name="s")
    return pl.kernel(
        _sc_body,
        mesh=mesh,
        compiler_params=pltpu.CompilerParams(
            needs_layout_passes=False, use_tc_tiling_on_sc=False),
        out_type=jax.ShapeDtypeStruct((BATCH,), jnp.float32),
        scratch_types=[
            pltpu.VMEM((SPW, L), jnp.int32),
            pltpu.VMEM((L, M), jnp.float32),
            pltpu.VMEM((L, M), jnp.float32),
            pltpu.VMEM((256,), jnp.float32),
            pltpu.VMEM((SPW,), jnp.float32),
            pltpu.SemaphoreType.DMA,
            pltpu.SemaphoreType.DMA,
        ],
    )(table, idx)


def kernel(inputs, epsilon):
    table = jnp.transpose(epsilon, (0, 2, 3, 4, 1)).reshape(ROWS, M)
    return _seg_gps(table, inputs.astype(jnp.int32))


# speedup vs baseline: 1.6063x; 1.0050x over previous
import functools

import jax
import jax.numpy as jnp
from jax import lax
from jax.experimental import pallas as pl
from jax.experimental.pallas import tpu as pltpu
from jax.experimental.pallas import tpu_sc as plsc

L = 64
M = 64
BATCH = 4096
NUP = 33
ROWS = 2 * L * NUP * NUP
S_STRIDE = L * NUP * NUP
I_STRIDE = NUP * NUP + 1
NU_STRIDE = NUP - 1

_NC, _NS, _NL = 2, 16, 16
NW = _NC * _NS
SPW = BATCH // NW
GRP = SPW // 16


def _idx_body(in_ref, idx_ref):
    s = in_ref[...].astype(jnp.float32)
    row = lax.broadcasted_iota(jnp.int32, (L, L), 0)
    col = lax.broadcasted_iota(jnp.int32, (L, L), 1)
    tri = (row < col).astype(jnp.float32)
    nu = jax.lax.dot(s, tri, precision=jax.lax.Precision.HIGHEST)
    site = lax.broadcasted_iota(jnp.int32, (BATCH, L), 1).astype(jnp.float32)
    idx = s * S_STRIDE + site * I_STRIDE + nu * NU_STRIDE
    idx_ref[...] = idx.astype(jnp.int32)


PAIRW = 2 * L
NPAIR = SPW // 2


def _sc_body(table_hbm, idx_hbm, out_hbm, idx_v, rows0, rows1, tmp_v, out_v,
             sem0, sem1):
    wid = lax.axis_index("s") * _NC + lax.axis_index("c")
    base = wid * SPW * L
    pltpu.sync_copy(idx_hbm.at[pl.ds(base, SPW * L)], idx_v)
    iota = lax.iota(jnp.int32, 16)

    def product(rows_v, off):
        def prod(j, accs):
            accs = list(accs)
            for r in range(8):
                row = off + 8 * j + r
                c = (r % 2) * 4
                for k in range(4):
                    accs[c + k] = accs[c + k] * rows_v[row, pl.ds(16 * k, 16)]
            return tuple(accs)

        ones = jnp.ones((16,), jnp.float32)
        accs = lax.fori_loop(0, L // 8, prod, (ones,) * 8)
        return (accs[0] * accs[4] + accs[1] * accs[5]
                + accs[2] * accs[6] + accs[3] * accs[7])

    def gather_pair(p, dst, sem):
        return pltpu.async_copy(
            table_hbm.at[idx_v.at[pl.ds(p * PAIRW, PAIRW)]], dst, sem)

    def wait_pair(p, dst, sem):
        pltpu.make_async_copy(
            table_hbm.at[idx_v.at[pl.ds(p * PAIRW, PAIRW)]], dst, sem).wait()

    gather_pair(0, rows0, sem0)

    def group(g, _):
        def quad(qq, _):
            p0 = g * 8 + 2 * qq
            s0 = 4 * qq
            gather_pair(p0 + 1, rows1, sem1)
            wait_pair(p0, rows0, sem0)
            tot_a = product(rows0, 0)
            tot_b = product(rows0, L)

            @pl.when(p0 < NPAIR - 2)
            def _():
                gather_pair(p0 + 2, rows0, sem0)

            wait_pair(p0 + 1, rows1, sem1)
            tot_c = product(rows1, 0)
            tot_d = product(rows1, L)
            tmp_v[pl.ds(s0 * 16, 16)] = tot_a
            tmp_v[pl.ds((s0 + 1) * 16, 16)] = tot_b
            tmp_v[pl.ds((s0 + 2) * 16, 16)] = tot_c
            tmp_v[pl.ds((s0 + 3) * 16, 16)] = tot_d
            return 0

        lax.fori_loop(0, 4, quad, 0)
        acc = jnp.zeros((16,), jnp.float32)
        for j in range(16):
            acc = acc + plsc.load_gather(tmp_v, [iota * 16 + j])
        out_v[pl.ds(g * 16, 16)] = acc
        return 0

    lax.fori_loop(0, GRP, group, 0)
    pltpu.sync_copy(out_v, out_hbm.at[pl.ds(wid * SPW, SPW)])


@jax.jit
def _seg_gps(table, inputs_i32):
    idx = pl.pallas_call(
        _idx_body,
        out_shape=jax.ShapeDtypeStruct((BATCH, L), jnp.int32),
    )(inputs_i32)
    mesh = plsc.VectorSubcoreMesh(core_axis_name="c", subcore_axis_name="s")
    return pl.kernel(
        _sc_body,
        mesh=mesh,
        compiler_params=pltpu.CompilerParams(
            needs_layout_passes=False, use_tc_tiling_on_sc=False),
        out_type=jax.ShapeDtypeStruct((BATCH,), jnp.float32),
        scratch_types=[
            pltpu.VMEM((SPW * L,), jnp.int32),
            pltpu.VMEM((PAIRW, M), jnp.float32),
            pltpu.VMEM((PAIRW, M), jnp.float32),
            pltpu.VMEM((256,), jnp.float32),
            pltpu.VMEM((SPW,), jnp.float32),
            pltpu.SemaphoreType.DMA,
            pltpu.SemaphoreType.DMA,
        ],
    )(table, idx.reshape(-1))


def kernel(inputs, epsilon):
    table = jnp.transpose(epsilon, (0, 2, 3, 4, 1)).reshape(ROWS, M)
    return _seg_gps(table, inputs.astype(jnp.int32))
